# baseline (speedup 1.0000x reference)
import math
import functools

import jax
import jax.numpy as jnp
from jax import lax
from jax.experimental import pallas as pl
from jax.experimental.pallas import tpu as pltpu
from jax.experimental.pallas import tpu_sc as plsc

VOCAB = 1000000
EMBED = 64
D_MODEL = 128
B = 4096
L = 200

NC = 2
NS = 16
NW = NC * NS

N = B * L
R = N // NW
XR = R // L
BLOCK = 12800
HALF = BLOCK // 2
NBLK = R // BLOCK
CHUNK = 2 * L
NCHUNK = HALF // CHUNK
SUBS = ((0, 0, 128), (128, 128, 72), (200, 0, 128), (328, 128, 72))


def _sc_gather_packed(x, table):
    mesh = plsc.VectorSubcoreMesh(core_axis_name="c", subcore_axis_name="s")

    @functools.partial(
        pl.kernel,
        out_type=jax.ShapeDtypeStruct((N // 2, 2 * EMBED), jnp.float32),
        mesh=mesh,
        scratch_types=[
            pltpu.VMEM((XR, 256), jnp.int32),
            pltpu.VMEM((CHUNK, EMBED), jnp.float32),
            pltpu.SemaphoreType.DMA,
        ],
        compiler_params=pltpu.CompilerParams(use_tc_tiling_on_sc=False),
    )
    def k(idx_hbm, table_hbm, emb_hbm, idx_v, rows_v, sem):
        wid = lax.axis_index("s") * NC + lax.axis_index("c")
        row_base = wid * (R // 2)

        pltpu.sync_copy(idx_hbm.at[pl.ds(wid * XR, XR)], idx_v)

        def chunk_body(t, carry):
            blk = t // (2 * NCHUNK)
            h = (t // NCHUNK) % 2
            c = t % NCHUNK
            r0 = blk * (2 * NBLK * NCHUNK) + h * (2 * NCHUNK) + c * 2
            descs = [
                pltpu.async_copy(
                    table_hbm.at[idx_v.at[r0 + dr, pl.ds(co, n)]],
                    rows_v.at[pl.ds(do, n)],
                    sem,
                )
                for do, co, n in SUBS
                for dr in (do // L,)
            ]
            for d in descs:
                d.wait()
            pltpu.sync_copy(
                rows_v,
                emb_hbm.at[
                    pl.ds(row_base + blk * HALF + c * CHUNK, CHUNK),
                    pl.ds(h * EMBED, EMBED),
                ],
            )
            return carry

        lax.fori_loop(0, NBLK * 2 * NCHUNK, chunk_body, 0)

    return k(x, table)


def _tc_matmul(emb2, Wa, Wb, b2):
    def body(emb_ref, wa_ref, wb_ref, b_ref, out_ref):
        e = emb_ref[...]
        top = jnp.dot(e, wa_ref[...], preferred_element_type=jnp.float32)
        bot = jnp.dot(e, wb_ref[...], preferred_element_type=jnp.float32)
        top = top + b_ref[...]
        bot = bot + b_ref[...]
        half_rows = BLOCK // L // 2
        out_ref[0:half_rows] = top.reshape(half_rows, L, D_MODEL)
        out_ref[half_rows:] = bot.reshape(half_rows, L, D_MODEL)

    return pl.pallas_call(
        body,
        grid=(N // BLOCK,),
        in_specs=[
            pl.BlockSpec((HALF, 2 * EMBED), lambda i: (i, 0)),
            pl.BlockSpec((2 * EMBED, D_MODEL), lambda i: (0, 0)),
            pl.BlockSpec((2 * EMBED, D_MODEL), lambda i: (0, 0)),
            pl.BlockSpec((1, D_MODEL), lambda i: (0, 0)),
        ],
        out_specs=pl.BlockSpec((BLOCK // L, L, D_MODEL), lambda i: (i, 0, 0)),
        out_shape=jax.ShapeDtypeStruct((B, L, D_MODEL), jnp.float32),
    )(emb2, Wa, Wb, b2)


def kernel(x, table, W, b):
    scale = math.sqrt(D_MODEL)
    xp = jnp.pad(x.astype(jnp.int32), ((0, 0), (0, 256 - L)))
    emb2 = _sc_gather_packed(xp, table)
    Ws = W * scale
    zero = jnp.zeros_like(Ws)
    Wa = jnp.concatenate([Ws, zero], axis=0)
    Wb = jnp.concatenate([zero, Ws], axis=0)
    b2 = (b * scale).reshape(1, D_MODEL)
    return _tc_matmul(emb2, Wa, Wb, b2)


# speedup vs baseline: 1.0410x; 1.0410x over previous
import math
import functools

import jax
import jax.numpy as jnp
from jax import lax
from jax.experimental import pallas as pl
from jax.experimental.pallas import tpu as pltpu
from jax.experimental.pallas import tpu_sc as plsc

VOCAB = 1000000
EMBED = 64
D_MODEL = 128
B = 4096
L = 200

NC = 2
NS = 16
NW = NC * NS

N = B * L
R = N // NW
XR = R // L
BLOCK = 12800
HALF = BLOCK // 2
NBLK = R // BLOCK
CHUNK = 2 * L
NCHUNK = HALF // CHUNK
SUBS = ((0, 0, 128), (128, 128, 72), (200, 0, 128), (328, 128, 72))
MMU = 2


def _sc_gather_packed(x, table):
    mesh = plsc.VectorSubcoreMesh(core_axis_name="c", subcore_axis_name="s")

    @functools.partial(
        pl.kernel,
        out_type=jax.ShapeDtypeStruct((N // 2, 2 * EMBED), jnp.float32),
        mesh=mesh,
        scratch_types=[
            pltpu.VMEM((XR, 256), jnp.int32),
            pltpu.VMEM((CHUNK, EMBED), jnp.float32),
            pltpu.VMEM((CHUNK, EMBED), jnp.float32),
            pltpu.SemaphoreType.DMA,
        ],
        compiler_params=pltpu.CompilerParams(use_tc_tiling_on_sc=False),
    )
    def k(idx_hbm, table_hbm, emb_hbm, idx_v, rows_v0, rows_v1, sem):
        wid = lax.axis_index("s") * NC + lax.axis_index("c")
        row_base = wid * (R // 2)

        pltpu.sync_copy(idx_hbm.at[pl.ds(wid * XR, XR)], idx_v)

        def fire(t, buf):
            r0 = t * 2
            return [
                pltpu.async_copy(
                    table_hbm.at[idx_v.at[r0 + do // L, pl.ds(co, n)]],
                    buf.at[pl.ds(do, n)],
                    sem,
                )
                for do, co, n in SUBS
            ]

        def flush(t, buf):
            blk = t // (2 * NCHUNK)
            h = (t // NCHUNK) % 2
            c = t % NCHUNK
            pltpu.sync_copy(
                buf,
                emb_hbm.at[
                    pl.ds(row_base + blk * HALF + c * CHUNK, CHUNK),
                    pl.ds(h * EMBED, EMBED),
                ],
            )

        def pair_body(u, carry):
            descs = fire(2 * u, rows_v0)

            @pl.when(u > 0)
            def _():
                flush(2 * u - 1, rows_v1)

            for d in descs:
                d.wait()
            descs = fire(2 * u + 1, rows_v1)
            flush(2 * u, rows_v0)
            for d in descs:
                d.wait()
            return carry

        lax.fori_loop(0, NBLK * NCHUNK, pair_body, 0)
        flush(2 * NBLK * NCHUNK - 1, rows_v1)

    return k(x, table)


def _tc_matmul(emb2, Wa, Wb, b2):
    def body(emb_ref, wa_ref, wb_ref, b_ref, out_ref):
        half_rows = BLOCK // L // 2
        for u in range(MMU):
            e = emb_ref[pl.ds(u * HALF, HALF), :]
            top = jnp.dot(e, wa_ref[...], preferred_element_type=jnp.float32)
            bot = jnp.dot(e, wb_ref[...], preferred_element_type=jnp.float32)
            top = top + b_ref[...]
            bot = bot + b_ref[...]
            r0 = u * (BLOCK // L)
            out_ref[r0:r0 + half_rows] = top.reshape(half_rows, L, D_MODEL)
            out_ref[r0 + half_rows:r0 + 2 * half_rows] = bot.reshape(
                half_rows, L, D_MODEL)

    return pl.pallas_call(
        body,
        grid=(N // (MMU * BLOCK),),
        in_specs=[
            pl.BlockSpec((MMU * HALF, 2 * EMBED), lambda i: (i, 0)),
            pl.BlockSpec((2 * EMBED, D_MODEL), lambda i: (0, 0)),
            pl.BlockSpec((2 * EMBED, D_MODEL), lambda i: (0, 0)),
            pl.BlockSpec((1, D_MODEL), lambda i: (0, 0)),
        ],
        out_specs=pl.BlockSpec(
            (MMU * BLOCK // L, L, D_MODEL), lambda i: (i, 0, 0)),
        out_shape=jax.ShapeDtypeStruct((B, L, D_MODEL), jnp.float32),
    )(emb2, Wa, Wb, b2)


def kernel(x, table, W, b):
    scale = math.sqrt(D_MODEL)
    xp = jnp.pad(x.astype(jnp.int32), ((0, 0), (0, 256 - L)))
    emb2 = _sc_gather_packed(xp, table)
    Ws = W * scale
    zero = jnp.zeros_like(Ws)
    Wa = jnp.concatenate([Ws, zero], axis=0)
    Wb = jnp.concatenate([zero, Ws], axis=0)
    b2 = (b * scale).reshape(1, D_MODEL)
    return _tc_matmul(emb2, Wa, Wb, b2)


# speedup vs baseline: 1.0420x; 1.0009x over previous
import math
import functools

import jax
import jax.numpy as jnp
from jax import lax
from jax.experimental import pallas as pl
from jax.experimental.pallas import tpu as pltpu
from jax.experimental.pallas import tpu_sc as plsc

VOCAB = 1000000
EMBED = 64
D_MODEL = 128
B = 4096
L = 200

NC = 2
NS = 16
NW = NC * NS

N = B * L
R = N // NW
XR = R // L
BLOCK = 12800
HALF = BLOCK // 2
NBLK = R // BLOCK
CHUNK = 2 * L
NCHUNK = HALF // CHUNK
SUBS = ((0, 0, 128), (128, 128, 72), (200, 0, 128), (328, 128, 72))
MMU = 2


def _sc_gather_packed(x, table):
    mesh = plsc.VectorSubcoreMesh(core_axis_name="c", subcore_axis_name="s")

    @functools.partial(
        pl.kernel,
        out_type=jax.ShapeDtypeStruct((N // 2, 2 * EMBED), jnp.float32),
        mesh=mesh,
        scratch_types=[
            pltpu.VMEM((XR, 256), jnp.int32),
            pltpu.VMEM((CHUNK, EMBED), jnp.float32),
            pltpu.VMEM((CHUNK, EMBED), jnp.float32),
            pltpu.SemaphoreType.DMA,
        ],
        compiler_params=pltpu.CompilerParams(use_tc_tiling_on_sc=False),
    )
    def k(idx_hbm, table_hbm, emb_hbm, idx_v, rows_v0, rows_v1, sem):
        wid = lax.axis_index("s") * NC + lax.axis_index("c")
        row_base = wid * (R // 2)

        pltpu.sync_copy(idx_hbm.at[pl.ds(wid * XR, XR)], idx_v)

        def fire(t, buf):
            r0 = t * 2
            return [
                pltpu.async_copy(
                    table_hbm.at[idx_v.at[r0 + do // L, pl.ds(co, n)]],
                    buf.at[pl.ds(do, n)],
                    sem,
                )
                for do, co, n in SUBS
            ]

        def flush(t, buf):
            blk = t // (2 * NCHUNK)
            h = (t // NCHUNK) % 2
            c = t % NCHUNK
            pltpu.sync_copy(
                buf,
                emb_hbm.at[
                    pl.ds(row_base + blk * HALF + c * CHUNK, CHUNK),
                    pl.ds(h * EMBED, EMBED),
                ],
            )

        def pair_body(u, carry):
            descs = fire(2 * u, rows_v0)

            @pl.when(u > 0)
            def _():
                flush(2 * u - 1, rows_v1)

            for d in descs:
                d.wait()
            descs = fire(2 * u + 1, rows_v1)
            flush(2 * u, rows_v0)
            for d in descs:
                d.wait()
            return carry

        lax.fori_loop(0, NBLK * NCHUNK, pair_body, 0)
        flush(2 * NBLK * NCHUNK - 1, rows_v1)

    return k(x, table)


def _tc_matmul(emb2, Wa, Wb, b2):
    def body(emb_ref, wa_ref, wb_ref, b_ref, out_ref):
        for u in range(MMU):
            e = emb_ref[pl.ds(u * HALF, HALF), :]
            top = jnp.dot(e, wa_ref[...], preferred_element_type=jnp.float32)
            bot = jnp.dot(e, wb_ref[...], preferred_element_type=jnp.float32)
            r0 = u * BLOCK
            out_ref[pl.ds(r0, HALF), :] = top + b_ref[...]
            out_ref[pl.ds(r0 + HALF, HALF), :] = bot + b_ref[...]

    return pl.pallas_call(
        body,
        grid=(N // (MMU * BLOCK),),
        in_specs=[
            pl.BlockSpec((MMU * HALF, 2 * EMBED), lambda i: (i, 0)),
            pl.BlockSpec((2 * EMBED, D_MODEL), lambda i: (0, 0)),
            pl.BlockSpec((2 * EMBED, D_MODEL), lambda i: (0, 0)),
            pl.BlockSpec((1, D_MODEL), lambda i: (0, 0)),
        ],
        out_specs=pl.BlockSpec((MMU * BLOCK, D_MODEL), lambda i: (i, 0)),
        out_shape=jax.ShapeDtypeStruct((N, D_MODEL), jnp.float32),
    )(emb2, Wa, Wb, b2)


def kernel(x, table, W, b):
    scale = math.sqrt(D_MODEL)
    xp = jnp.pad(x.astype(jnp.int32), ((0, 0), (0, 256 - L)))
    emb2 = _sc_gather_packed(xp, table)
    Ws = W * scale
    zero = jnp.zeros_like(Ws)
    Wa = jnp.concatenate([Ws, zero], axis=0)
    Wb = jnp.concatenate([zero, Ws], axis=0)
    b2 = (b * scale).reshape(1, D_MODEL)
    return _tc_matmul(emb2, Wa, Wb, b2).reshape(B, L, D_MODEL)
